# Initial kernel scaffold; baseline (speedup 1.0000x reference)
import jax
import jax.numpy as jnp
from jax.experimental import pallas as pl


def kernel(x, edge_index, fc_W, fc_b, c1_Wl, c1_bl, c1_Wr, c2_Wl, c2_bl, c2_Wr, bn_Wl, bn_bl, bn_Wr, d1_Wl, d1_bl, d1_Wr, d2_Wl, d2_bl, d2_Wr, out_W, out_b, s1_W, s1_b, s2_W, s2_b):
    raise NotImplementedError("write your pallas kernel here")


# speedup vs baseline: 6.3564x; 6.3564x over previous
import functools

import jax
import jax.numpy as jnp
from jax import lax
from jax.experimental import pallas as pl
from jax.experimental.pallas import tpu as pltpu
from jax.experimental.pallas import tpu_sc as plsc

N = 10000
E = 320000
ROWB = 1000
WIN = 128
NWIN = E // WIN
NWORK = 32
ZCH = 640


def _dg(a, w):
    return lax.dot_general(
        a, w, (((1,), (1,)), ((), ())),
        preferred_element_type=jnp.float32,
        precision=lax.Precision.HIGHEST)


@functools.lru_cache(maxsize=None)
def _make_segsum(d):
    mesh = plsc.VectorSubcoreMesh(core_axis_name="c", subcore_axis_name="s")

    @functools.partial(
        pl.kernel,
        mesh=mesh,
        out_type=jax.ShapeDtypeStruct((2, N, d), jnp.float32),
        scratch_types=[
            pltpu.VMEM((WIN,), jnp.int32),
            pltpu.VMEM((WIN,), jnp.int32),
            pltpu.VMEM((WIN, d), jnp.float32),
            pltpu.VMEM_SHARED((N, d), jnp.float32),
            pltpu.SemaphoreType.DMA,
        ],
        compiler_params=pltpu.CompilerParams(use_tc_tiling_on_sc=False),
    )
    def segsum(x_hbm, src_hbm, dst_hbm, zr_hbm, out_hbm,
               sidx_v, didx_v, rows_v, acc_sh, sem):
        c = lax.axis_index("c")
        s = lax.axis_index("s")
        wid = s * 2 + c

        pltpu.sync_copy(zr_hbm, rows_v)

        @pl.when(s < 15)
        def _():
            for j in range(5):
                pltpu.sync_copy(rows_v, acc_sh.at[pl.ds(s * ZCH + j * WIN, WIN), :])

        @pl.when(s == 15)
        def _():
            for j in range(3):
                pltpu.sync_copy(rows_v, acc_sh.at[pl.ds(15 * ZCH + j * WIN, WIN), :])
            pltpu.sync_copy(rows_v.at[pl.ds(0, 16), :],
                            acc_sh.at[pl.ds(15 * ZCH + 3 * WIN, 16), :])

        plsc.subcore_barrier()

        nk = (NWIN - wid + (NWORK - 1)) // NWORK

        def body(k, carry):
            base = pl.multiple_of((wid + k * NWORK) * WIN, 8)
            pltpu.sync_copy(src_hbm.at[pl.ds(base, WIN)], sidx_v)
            pltpu.sync_copy(dst_hbm.at[pl.ds(base, WIN)], didx_v)
            pltpu.async_copy(x_hbm.at[sidx_v], rows_v, sem).wait()
            pltpu.sync_copy(rows_v, acc_sh.at[didx_v], add=True)
            return carry

        lax.fori_loop(0, nk, body, 0)

        plsc.subcore_barrier()

        @pl.when(s < 15)
        def _():
            for j in range(5):
                r0 = s * ZCH + j * WIN
                pltpu.sync_copy(acc_sh.at[pl.ds(r0, WIN), :],
                                out_hbm.at[c, pl.ds(r0, WIN), :])

        @pl.when(s == 15)
        def _():
            for j in range(3):
                r0 = 15 * ZCH + j * WIN
                pltpu.sync_copy(acc_sh.at[pl.ds(r0, WIN), :],
                                out_hbm.at[c, pl.ds(r0, WIN), :])
            pltpu.sync_copy(acc_sh.at[pl.ds(15 * ZCH + 3 * WIN, 16), :],
                            out_hbm.at[c, pl.ds(15 * ZCH + 3 * WIN, 16), :])

    return segsum


def _segsum(xin, src, dst):
    d = xin.shape[1]
    zr = jnp.zeros((WIN, d), jnp.float32)
    return _make_segsum(d)(xin, src, dst, zr)


def _row_spec(d):
    return pl.BlockSpec((ROWB, d), lambda i: (i, 0))


def _seg_spec(d):
    return pl.BlockSpec((2, ROWB, d), lambda i: (0, i, 0))


_CNT_SPEC = pl.BlockSpec((2, 1, 1, ROWB), lambda i: (0, i, 0, 0))


def _full_spec(shape):
    rank = len(shape)
    return pl.BlockSpec(shape, lambda i: (0,) * rank)


def _tc_call(body, in_arrays, in_specs, out_d):
    if isinstance(out_d, (tuple, list)):
        out_shape = tuple(jax.ShapeDtypeStruct((N, dd), jnp.float32)
                          for dd in out_d)
        out_specs = tuple(_row_spec(dd) for dd in out_d)
    else:
        out_shape = jax.ShapeDtypeStruct((N, out_d), jnp.float32)
        out_specs = _row_spec(out_d)
    return pl.pallas_call(
        body,
        grid=(N // ROWB,),
        in_specs=in_specs,
        out_specs=out_specs,
        out_shape=out_shape,
    )(*in_arrays)


def _invc(cnt_ref):
    cnt = cnt_ref[0, 0, 0, :] + cnt_ref[1, 0, 0, :]
    return 1.0 / jnp.maximum(cnt, 1.0)


def _mean(seg_ref, cnt_ref):
    ssum = seg_ref[0] + seg_ref[1]
    return ssum * _invc(cnt_ref)[:, None]


def _stage_a(x_r, w_r, b_r, o_r):
    x0 = jnp.tanh(_dg(x_r[...], w_r[...]) + b_r[...])
    o_r[...] = jnp.concatenate(
        [x0, jnp.ones((ROWB, 16), jnp.float32)], axis=1)


def _stage_b(s_r, c_r, x_r, wl_r, bl_r, wr_r, o_r):
    ssum = s_r[0, :, :32] + s_r[1, :, :32]
    mean = ssum * _invc(c_r)[:, None]
    o_r[...] = jnp.tanh(_dg(mean, wl_r[...]) + bl_r[...] +
                        _dg(x_r[:, :32], wr_r[...]))


def _stage_sage(s_r, c_r, x_r, wl_r, bl_r, wr_r, o_r):
    mean = _mean(s_r, c_r)
    o_r[...] = jnp.tanh(_dg(mean, wl_r[...]) + bl_r[...] +
                        _dg(x_r[...], wr_r[...]))


def _stage_d(s_r, c_r, x2_r, bnwl_r, bnbl_r, bnwr_r, d1wl_r, x1_r, s1w_r,
             s1b_r, x3_r, g3_r, sk1_r):
    mean = _mean(s_r, c_r)
    x3 = jnp.tanh(_dg(mean, bnwl_r[...]) + bnbl_r[...] +
                  _dg(x2_r[...], bnwr_r[...]))
    x3_r[...] = x3
    g3_r[...] = _dg(x3, d1wl_r[:, :128]) + _dg(x2_r[...], d1wl_r[:, 128:])
    sk1_r[...] = _dg(x1_r[...], s1w_r[...]) + s1b_r[...]


def _stage_e(s_r, c_r, x3_r, x2_r, d1bl_r, d1wr_r, sk1_r, d2wl_r,
             h4_r, g4_r):
    mean = _mean(s_r, c_r)
    h4 = jnp.tanh(mean + d1bl_r[...] + _dg(x3_r[...], d1wr_r[:, :128]) +
                  _dg(x2_r[...], d1wr_r[:, 128:]))
    h4_r[...] = h4
    g4_r[...] = _dg(h4, d2wl_r[:, :64]) + _dg(sk1_r[...], d2wl_r[:, 64:])


def _stage_f(s_r, c_r, h4_r, sk1_r, d2bl_r, d2wr_r, x0_r, s2w_r, s2b_r,
             outw_r, outb_r, o_r):
    mean = _mean(s_r, c_r)
    h5 = jnp.tanh(mean + d2bl_r[...] + _dg(h4_r[...], d2wr_r[:, :64]) +
                  _dg(sk1_r[...], d2wr_r[:, 64:]))
    skip2 = _dg(x0_r[:, :32], s2w_r[...]) + s2b_r[...]
    o_r[...] = (_dg(h5, outw_r[:, :32]) + _dg(skip2, outw_r[:, 32:]) +
                outb_r[...])


def kernel(x, edge_index, fc_W, fc_b, c1_Wl, c1_bl, c1_Wr, c2_Wl, c2_bl,
           c2_Wr, bn_Wl, bn_bl, bn_Wr, d1_Wl, d1_bl, d1_Wr, d2_Wl, d2_bl,
           d2_Wr, out_W, out_b, s1_W, s1_b, s2_W, s2_b):
    src = edge_index[0]
    dst = edge_index[1]

    r = lambda b: b.reshape(1, -1)

    x0a = _tc_call(
        _stage_a, (x, fc_W, r(fc_b)),
        [_row_spec(128), _full_spec((32, 128)), _full_spec((1, 32))], 48)

    s0 = _segsum(x0a, src, dst)
    cnt = s0[:, :, 32].reshape(2, N // ROWB, 1, ROWB)
    x1 = _tc_call(
        _stage_b, (s0, cnt, x0a, c1_Wl, r(c1_bl), c1_Wr),
        [_seg_spec(48), _CNT_SPEC, _row_spec(48), _full_spec((64, 32)),
         _full_spec((1, 64)), _full_spec((64, 32))], 64)

    s1, = (_segsum(x1, src, dst),)
    x2 = _tc_call(
        _stage_sage, (s1, cnt, x1, c2_Wl, r(c2_bl), c2_Wr),
        [_seg_spec(64), _CNT_SPEC, _row_spec(64), _full_spec((128, 64)),
         _full_spec((1, 128)), _full_spec((128, 64))], 128)

    s2 = _segsum(x2, src, dst)
    x3, g3, skip1 = _tc_call(
        _stage_d, (s2, cnt, x2, bn_Wl, r(bn_bl), bn_Wr, d1_Wl, x1, s1_W,
                   r(s1_b)),
        [_seg_spec(128), _CNT_SPEC, _row_spec(128), _full_spec((128, 128)),
         _full_spec((1, 128)), _full_spec((128, 128)), _full_spec((64, 256)),
         _row_spec(64), _full_spec((64, 64)), _full_spec((1, 64))],
        (128, 64, 64))

    s3 = _segsum(g3, src, dst)
    h4, g4 = _tc_call(
        _stage_e, (s3, cnt, x3, x2, r(d1_bl), d1_Wr, skip1, d2_Wl),
        [_seg_spec(64), _CNT_SPEC, _row_spec(128), _row_spec(128),
         _full_spec((1, 64)), _full_spec((64, 256)), _row_spec(64),
         _full_spec((32, 128))], (64, 32))

    s4 = _segsum(g4, src, dst)
    o = _tc_call(
        _stage_f, (s4, cnt, h4, skip1, r(d2_bl), d2_Wr, x0a, s2_W, r(s2_b),
                   out_W, r(out_b)),
        [_seg_spec(32), _CNT_SPEC, _row_spec(64), _row_spec(64),
         _full_spec((1, 32)), _full_spec((32, 128)), _row_spec(48),
         _full_spec((32, 32)), _full_spec((1, 32)), _full_spec((3, 64)),
         _full_spec((1, 3))], 3)

    return o


# speedup vs baseline: 11.9347x; 1.8776x over previous
import functools

import jax
import jax.numpy as jnp
from jax import lax
from jax.experimental import pallas as pl
from jax.experimental.pallas import tpu as pltpu
from jax.experimental.pallas import tpu_sc as plsc

N = 10000
E = 320000
ROWB = 1000
WIN = 128
NWORK = 32
WPT = 80
EPAD = NWORK * WPT * WIN
ZCH = 640
NJUNK = 16


def _dg(a, w):
    return lax.dot_general(
        a, w, (((1,), (1,)), ((), ())),
        preferred_element_type=jnp.float32,
        precision=lax.Precision.HIGHEST)


@functools.lru_cache(maxsize=None)
def _make_segsum(d):
    mesh = plsc.VectorSubcoreMesh(core_axis_name="c", subcore_axis_name="s")
    assert d <= 64
    nbuf = 5 if d == 64 else 8
    ngrp = WPT // nbuf

    @functools.partial(
        pl.kernel,
        mesh=mesh,
        out_type=jax.ShapeDtypeStruct((2, N, d), jnp.float32),
        scratch_types=[
            pltpu.VMEM((WPT, WIN), jnp.int32),
            pltpu.VMEM((WPT, WIN), jnp.int32),
            pltpu.VMEM((nbuf, WIN, d), jnp.float32),
            pltpu.VMEM_SHARED((N + NJUNK, d), jnp.float32),
        ] + [pltpu.SemaphoreType.DMA] * (2 * nbuf),
        compiler_params=pltpu.CompilerParams(use_tc_tiling_on_sc=False),
    )
    def segsum(x_hbm, srcw_hbm, dstw_hbm, zr_hbm, out_hbm,
               sidx, didx, rows, acc_sh, *sems):
        sem_g = sems[:nbuf]
        sem_s = sems[nbuf:]
        c = lax.axis_index("c")
        s = lax.axis_index("s")
        wid = s * 2 + c

        pltpu.sync_copy(srcw_hbm.at[pl.ds(wid * WPT, WPT), :], sidx)
        pltpu.sync_copy(dstw_hbm.at[pl.ds(wid * WPT, WPT), :], didx)

        pltpu.sync_copy(zr_hbm, rows.at[0])

        def _zfire(r0, nrows):
            pltpu.async_copy(rows.at[0, pl.ds(0, nrows), :],
                             acc_sh.at[pl.ds(r0, nrows), :], sem_g[0])

        def _zdrain(r0, nrows):
            pltpu.make_async_copy(rows.at[0, pl.ds(0, nrows), :],
                                  acc_sh.at[pl.ds(r0, nrows), :],
                                  sem_g[0]).wait()

        @pl.when(s < 15)
        def _():
            for j in range(5):
                _zfire(s * ZCH + j * WIN, WIN)
            for j in range(5):
                _zdrain(s * ZCH + j * WIN, WIN)

        @pl.when(s == 15)
        def _():
            for j in range(3):
                _zfire(15 * ZCH + j * WIN, WIN)
            _zfire(15 * ZCH + 3 * WIN, 16)
            _zfire(N, NJUNK)
            for j in range(3):
                _zdrain(15 * ZCH + j * WIN, WIN)
            _zdrain(15 * ZCH + 3 * WIN, 16)
            _zdrain(N, NJUNK)

        plsc.subcore_barrier()

        def _gather(w, j):
            return pltpu.make_async_copy(x_hbm.at[sidx.at[w]], rows.at[j],
                                         sem_g[j])

        def _scatter(w, j):
            return pltpu.make_async_copy(rows.at[j], acc_sh.at[didx.at[w]],
                                         sem_s[j])

        def group(g, carry):
            wbase = g * nbuf
            for j in range(nbuf):
                @pl.when(g > 0)
                def _():
                    _scatter(wbase + j, j).wait()
                _gather(wbase + j, j).start()
            for j in range(nbuf):
                _gather(wbase + j, j).wait()
                _scatter(wbase + j, j).start(add=True)
            return carry

        lax.fori_loop(0, ngrp, group, 0)
        for j in range(nbuf):
            _scatter(j, j).wait()

        plsc.subcore_barrier()

        def _dfire(r0, nrows):
            pltpu.async_copy(acc_sh.at[pl.ds(r0, nrows), :],
                             out_hbm.at[c, pl.ds(r0, nrows), :], sem_s[0])

        def _ddrain(r0, nrows):
            pltpu.make_async_copy(acc_sh.at[pl.ds(r0, nrows), :],
                                  out_hbm.at[c, pl.ds(r0, nrows), :],
                                  sem_s[0]).wait()

        @pl.when(s < 15)
        def _():
            for j in range(5):
                _dfire(s * ZCH + j * WIN, WIN)
            for j in range(5):
                _ddrain(s * ZCH + j * WIN, WIN)

        @pl.when(s == 15)
        def _():
            for j in range(3):
                _dfire(15 * ZCH + j * WIN, WIN)
            _dfire(15 * ZCH + 3 * WIN, 16)
            for j in range(3):
                _ddrain(15 * ZCH + j * WIN, WIN)
            _ddrain(15 * ZCH + 3 * WIN, 16)

    return segsum


def _segsum(xin, srcw, dstw):
    d = xin.shape[1]
    zr = jnp.zeros((WIN, d), jnp.float32)
    return _make_segsum(d)(xin, srcw, dstw, zr)


def _row_spec(d):
    return pl.BlockSpec((ROWB, d), lambda i: (i, 0))


def _seg_spec(d):
    return pl.BlockSpec((2, ROWB, d), lambda i: (0, i, 0))


_CNT_SPEC = pl.BlockSpec((2, 1, 1, ROWB), lambda i: (0, i, 0, 0))


def _full_spec(shape):
    rank = len(shape)
    return pl.BlockSpec(shape, lambda i: (0,) * rank)


def _tc_call(body, in_arrays, in_specs, out_d):
    if isinstance(out_d, (tuple, list)):
        out_shape = tuple(jax.ShapeDtypeStruct((N, dd), jnp.float32)
                          for dd in out_d)
        out_specs = tuple(_row_spec(dd) for dd in out_d)
    else:
        out_shape = jax.ShapeDtypeStruct((N, out_d), jnp.float32)
        out_specs = _row_spec(out_d)
    return pl.pallas_call(
        body,
        grid=(N // ROWB,),
        in_specs=in_specs,
        out_specs=out_specs,
        out_shape=out_shape,
    )(*in_arrays)


def _invc(cnt_ref):
    cnt = cnt_ref[0, 0, 0, :] + cnt_ref[1, 0, 0, :]
    return 1.0 / jnp.maximum(cnt, 1.0)


def _mean(seg_ref, cnt_ref):
    ssum = seg_ref[0] + seg_ref[1]
    return ssum * _invc(cnt_ref)[:, None]


def _stage_a(x_r, w_r, b_r, o_r):
    x0 = jnp.tanh(_dg(x_r[...], w_r[...]) + b_r[...])
    o_r[...] = jnp.concatenate(
        [x0, jnp.ones((ROWB, 16), jnp.float32)], axis=1)


def _stage_b(s_r, c_r, x_r, wl_r, bl_r, wr_r, o_r):
    ssum = s_r[0, :, :32] + s_r[1, :, :32]
    mean = ssum * _invc(c_r)[:, None]
    o_r[...] = jnp.tanh(_dg(mean, wl_r[...]) + bl_r[...] +
                        _dg(x_r[:, :32], wr_r[...]))


def _stage_sage(s_r, c_r, x_r, wl_r, bl_r, wr_r, o_r):
    mean = _mean(s_r, c_r)
    o_r[...] = jnp.tanh(_dg(mean, wl_r[...]) + bl_r[...] +
                        _dg(x_r[...], wr_r[...]))


def _stage_d(sa_r, sb_r, c_r, x2_r, bnwl_r, bnbl_r, bnwr_r, d1wl_r, x1_r,
             s1w_r, s1b_r, x3_r, g3_r, sk1_r):
    invc = _invc(c_r)[:, None]
    mean_a = (sa_r[0] + sa_r[1]) * invc
    mean_b = (sb_r[0] + sb_r[1]) * invc
    x3 = jnp.tanh(_dg(mean_a, bnwl_r[:, :64]) + _dg(mean_b, bnwl_r[:, 64:]) +
                  bnbl_r[...] + _dg(x2_r[...], bnwr_r[...]))
    x3_r[...] = x3
    g3_r[...] = _dg(x3, d1wl_r[:, :128]) + _dg(x2_r[...], d1wl_r[:, 128:])
    sk1_r[...] = _dg(x1_r[...], s1w_r[...]) + s1b_r[...]


def _stage_e(s_r, c_r, x3_r, x2_r, d1bl_r, d1wr_r, sk1_r, d2wl_r,
             h4_r, g4_r):
    mean = _mean(s_r, c_r)
    h4 = jnp.tanh(mean + d1bl_r[...] + _dg(x3_r[...], d1wr_r[:, :128]) +
                  _dg(x2_r[...], d1wr_r[:, 128:]))
    h4_r[...] = h4
    g4_r[...] = _dg(h4, d2wl_r[:, :64]) + _dg(sk1_r[...], d2wl_r[:, 64:])


def _stage_f(s_r, c_r, h4_r, sk1_r, d2bl_r, d2wr_r, x0_r, s2w_r, s2b_r,
             outw_r, outb_r, o_r):
    mean = _mean(s_r, c_r)
    h5 = jnp.tanh(mean + d2bl_r[...] + _dg(h4_r[...], d2wr_r[:, :64]) +
                  _dg(sk1_r[...], d2wr_r[:, 64:]))
    skip2 = _dg(x0_r[:, :32], s2w_r[...]) + s2b_r[...]
    o_r[...] = (_dg(h5, outw_r[:, :32]) + _dg(skip2, outw_r[:, 32:]) +
                outb_r[...])


def kernel(x, edge_index, fc_W, fc_b, c1_Wl, c1_bl, c1_Wr, c2_Wl, c2_bl,
           c2_Wr, bn_Wl, bn_bl, bn_Wr, d1_Wl, d1_bl, d1_Wr, d2_Wl, d2_bl,
           d2_Wr, out_W, out_b, s1_W, s1_b, s2_W, s2_b):
    pad = EPAD - E
    pada = jnp.arange(pad, dtype=jnp.int32)
    srcw = jnp.concatenate([edge_index[0], pada % 256]).reshape(-1, WIN)
    dstw = jnp.concatenate([edge_index[1], N + (pada % NJUNK)]).reshape(-1, WIN)

    r = lambda b: b.reshape(1, -1)

    x0a = _tc_call(
        _stage_a, (x, fc_W, r(fc_b)),
        [_row_spec(128), _full_spec((32, 128)), _full_spec((1, 32))], 48)

    s0 = _segsum(x0a, srcw, dstw)
    cnt = s0[:, :, 32].reshape(2, N // ROWB, 1, ROWB)
    x1 = _tc_call(
        _stage_b, (s0, cnt, x0a, c1_Wl, r(c1_bl), c1_Wr),
        [_seg_spec(48), _CNT_SPEC, _row_spec(48), _full_spec((64, 32)),
         _full_spec((1, 64)), _full_spec((64, 32))], 64)

    s1, = (_segsum(x1, srcw, dstw),)
    x2 = _tc_call(
        _stage_sage, (s1, cnt, x1, c2_Wl, r(c2_bl), c2_Wr),
        [_seg_spec(64), _CNT_SPEC, _row_spec(64), _full_spec((128, 64)),
         _full_spec((1, 128)), _full_spec((128, 64))], 128)

    s2a = _segsum(x2[:, :64], srcw, dstw)
    s2b = _segsum(x2[:, 64:], srcw, dstw)
    x3, g3, skip1 = _tc_call(
        _stage_d, (s2a, s2b, cnt, x2, bn_Wl, r(bn_bl), bn_Wr, d1_Wl, x1,
                   s1_W, r(s1_b)),
        [_seg_spec(64), _seg_spec(64), _CNT_SPEC, _row_spec(128),
         _full_spec((128, 128)), _full_spec((1, 128)),
         _full_spec((128, 128)), _full_spec((64, 256)),
         _row_spec(64), _full_spec((64, 64)), _full_spec((1, 64))],
        (128, 64, 64))

    s3 = _segsum(g3, srcw, dstw)
    h4, g4 = _tc_call(
        _stage_e, (s3, cnt, x3, x2, r(d1_bl), d1_Wr, skip1, d2_Wl),
        [_seg_spec(64), _CNT_SPEC, _row_spec(128), _row_spec(128),
         _full_spec((1, 64)), _full_spec((64, 256)), _row_spec(64),
         _full_spec((32, 128))], (64, 32))

    s4 = _segsum(g4, srcw, dstw)
    o = _tc_call(
        _stage_f, (s4, cnt, h4, skip1, r(d2_bl), d2_Wr, x0a, s2_W, r(s2_b),
                   out_W, r(out_b)),
        [_seg_spec(32), _CNT_SPEC, _row_spec(64), _row_spec(64),
         _full_spec((1, 32)), _full_spec((32, 128)), _row_spec(48),
         _full_spec((32, 32)), _full_spec((1, 32)), _full_spec((3, 64)),
         _full_spec((1, 3))], 3)

    return o


# speedup vs baseline: 15.6675x; 1.3128x over previous
import functools

import jax
import jax.numpy as jnp
from jax import lax
from jax.experimental import pallas as pl
from jax.experimental.pallas import tpu as pltpu
from jax.experimental.pallas import tpu_sc as plsc

N = 10000
E = 320000
ROWB = 1000
WIN = 128
NWORK = 32
WPT = 80
EPAD = NWORK * WPT * WIN
ZCH = 640
NJUNK = 16


def _dg(a, w):
    return lax.dot_general(
        a, w, (((1,), (1,)), ((), ())),
        preferred_element_type=jnp.float32)


@functools.lru_cache(maxsize=None)
def _make_segsum(d):
    mesh = plsc.VectorSubcoreMesh(core_axis_name="c", subcore_axis_name="s")
    assert d <= 64
    nbuf = 5 if d == 64 else 8
    ngrp = WPT // nbuf

    @functools.partial(
        pl.kernel,
        mesh=mesh,
        out_type=jax.ShapeDtypeStruct((2, N, d), jnp.float32),
        scratch_types=[
            pltpu.VMEM((WPT, WIN), jnp.int32),
            pltpu.VMEM((WPT, WIN), jnp.int32),
            pltpu.VMEM((nbuf, WIN, d), jnp.float32),
            pltpu.VMEM_SHARED((N + NJUNK, d), jnp.float32),
        ] + [pltpu.SemaphoreType.DMA] * (2 * nbuf),
        compiler_params=pltpu.CompilerParams(use_tc_tiling_on_sc=False),
    )
    def segsum(x_hbm, srcw_hbm, dstw_hbm, zr_hbm, out_hbm,
               sidx, didx, rows, acc_sh, *sems):
        sem_g = sems[:nbuf]
        sem_s = sems[nbuf:]
        c = lax.axis_index("c")
        s = lax.axis_index("s")
        wid = s * 2 + c

        pltpu.sync_copy(srcw_hbm.at[pl.ds(wid * WPT, WPT), :], sidx)
        pltpu.sync_copy(dstw_hbm.at[pl.ds(wid * WPT, WPT), :], didx)

        pltpu.sync_copy(zr_hbm, rows.at[0])

        def _zfire(r0, nrows):
            pltpu.async_copy(rows.at[0, pl.ds(0, nrows), :],
                             acc_sh.at[pl.ds(r0, nrows), :], sem_g[0])

        def _zdrain(r0, nrows):
            pltpu.make_async_copy(rows.at[0, pl.ds(0, nrows), :],
                                  acc_sh.at[pl.ds(r0, nrows), :],
                                  sem_g[0]).wait()

        @pl.when(s < 15)
        def _():
            for j in range(5):
                _zfire(s * ZCH + j * WIN, WIN)
            for j in range(5):
                _zdrain(s * ZCH + j * WIN, WIN)

        @pl.when(s == 15)
        def _():
            for j in range(3):
                _zfire(15 * ZCH + j * WIN, WIN)
            _zfire(15 * ZCH + 3 * WIN, 16)
            _zfire(N, NJUNK)
            for j in range(3):
                _zdrain(15 * ZCH + j * WIN, WIN)
            _zdrain(15 * ZCH + 3 * WIN, 16)
            _zdrain(N, NJUNK)

        plsc.subcore_barrier()

        def _gather(w, j):
            return pltpu.make_async_copy(x_hbm.at[sidx.at[w]], rows.at[j],
                                         sem_g[j])

        def _scatter(w, j):
            return pltpu.make_async_copy(rows.at[j], acc_sh.at[didx.at[w]],
                                         sem_s[j])

        def group(g, carry):
            wbase = g * nbuf
            for j in range(nbuf):
                @pl.when(g > 0)
                def _():
                    _scatter(wbase + j, j).wait()
                _gather(wbase + j, j).start()
            for j in range(nbuf):
                _gather(wbase + j, j).wait()
                _scatter(wbase + j, j).start(add=True)
            return carry

        lax.fori_loop(0, ngrp, group, 0)
        for j in range(nbuf):
            _scatter(j, j).wait()

        plsc.subcore_barrier()

        def _dfire(r0, nrows):
            pltpu.async_copy(acc_sh.at[pl.ds(r0, nrows), :],
                             out_hbm.at[c, pl.ds(r0, nrows), :], sem_s[0])

        def _ddrain(r0, nrows):
            pltpu.make_async_copy(acc_sh.at[pl.ds(r0, nrows), :],
                                  out_hbm.at[c, pl.ds(r0, nrows), :],
                                  sem_s[0]).wait()

        @pl.when(s < 15)
        def _():
            for j in range(5):
                _dfire(s * ZCH + j * WIN, WIN)
            for j in range(5):
                _ddrain(s * ZCH + j * WIN, WIN)

        @pl.when(s == 15)
        def _():
            for j in range(3):
                _dfire(15 * ZCH + j * WIN, WIN)
            _dfire(15 * ZCH + 3 * WIN, 16)
            for j in range(3):
                _ddrain(15 * ZCH + j * WIN, WIN)
            _ddrain(15 * ZCH + 3 * WIN, 16)

    return segsum


def _segsum(xin, srcw, dstw):
    d = xin.shape[1]
    zr = jnp.zeros((WIN, d), jnp.float32)
    return _make_segsum(d)(xin, srcw, dstw, zr)


@functools.lru_cache(maxsize=None)
def _make_segsum_colsplit():
    d = 64
    mesh = plsc.VectorSubcoreMesh(core_axis_name="c", subcore_axis_name="s")
    nbuf = 5
    wpt = 2 * WPT
    ngrp = wpt // nbuf

    @functools.partial(
        pl.kernel,
        mesh=mesh,
        out_type=jax.ShapeDtypeStruct((2, N, d), jnp.float32),
        scratch_types=[
            pltpu.VMEM((wpt, WIN), jnp.int32),
            pltpu.VMEM((wpt, WIN), jnp.int32),
            pltpu.VMEM((nbuf, WIN, d), jnp.float32),
            pltpu.VMEM_SHARED((N + NJUNK, d), jnp.float32),
        ] + [pltpu.SemaphoreType.DMA] * (2 * nbuf),
        compiler_params=pltpu.CompilerParams(use_tc_tiling_on_sc=False),
    )
    def segsum(xa_hbm, xb_hbm, srcw_hbm, dstw_hbm, zr_hbm, out_hbm,
               sidx, didx, rows, acc_sh, *sems):
        sem_g = sems[:nbuf]
        sem_s = sems[nbuf:]
        c = lax.axis_index("c")
        s = lax.axis_index("s")

        pltpu.sync_copy(srcw_hbm.at[pl.ds(s * wpt, wpt), :], sidx)
        pltpu.sync_copy(dstw_hbm.at[pl.ds(s * wpt, wpt), :], didx)

        pltpu.sync_copy(zr_hbm, rows.at[0])

        def _zfire(r0, nrows):
            pltpu.async_copy(rows.at[0, pl.ds(0, nrows), :],
                             acc_sh.at[pl.ds(r0, nrows), :], sem_g[0])

        def _zdrain(r0, nrows):
            pltpu.make_async_copy(rows.at[0, pl.ds(0, nrows), :],
                                  acc_sh.at[pl.ds(r0, nrows), :],
                                  sem_g[0]).wait()

        @pl.when(s < 15)
        def _():
            for j in range(5):
                _zfire(s * ZCH + j * WIN, WIN)
            for j in range(5):
                _zdrain(s * ZCH + j * WIN, WIN)

        @pl.when(s == 15)
        def _():
            for j in range(3):
                _zfire(15 * ZCH + j * WIN, WIN)
            _zfire(15 * ZCH + 3 * WIN, 16)
            _zfire(N, NJUNK)
            for j in range(3):
                _zdrain(15 * ZCH + j * WIN, WIN)
            _zdrain(15 * ZCH + 3 * WIN, 16)
            _zdrain(N, NJUNK)

        plsc.subcore_barrier()

        def _gather(x_hbm, w, j):
            return pltpu.make_async_copy(x_hbm.at[sidx.at[w]], rows.at[j],
                                         sem_g[j])

        def _scatter(w, j):
            return pltpu.make_async_copy(rows.at[j], acc_sh.at[didx.at[w]],
                                         sem_s[j])

        def _loop(x_hbm):
            def group(g, carry):
                wbase = g * nbuf
                for j in range(nbuf):
                    @pl.when(g > 0)
                    def _():
                        _scatter(wbase + j, j).wait()
                    _gather(x_hbm, wbase + j, j).start()
                for j in range(nbuf):
                    _gather(x_hbm, wbase + j, j).wait()
                    _scatter(wbase + j, j).start(add=True)
                return carry
            lax.fori_loop(0, ngrp, group, 0)

        @pl.when(c == 0)
        def _():
            _loop(xa_hbm)

        @pl.when(c == 1)
        def _():
            _loop(xb_hbm)

        for j in range(nbuf):
            _scatter(j, j).wait()

        plsc.subcore_barrier()

        def _dfire(r0, nrows):
            pltpu.async_copy(acc_sh.at[pl.ds(r0, nrows), :],
                             out_hbm.at[c, pl.ds(r0, nrows), :], sem_s[0])

        def _ddrain(r0, nrows):
            pltpu.make_async_copy(acc_sh.at[pl.ds(r0, nrows), :],
                                  out_hbm.at[c, pl.ds(r0, nrows), :],
                                  sem_s[0]).wait()

        @pl.when(s < 15)
        def _():
            for j in range(5):
                _dfire(s * ZCH + j * WIN, WIN)
            for j in range(5):
                _ddrain(s * ZCH + j * WIN, WIN)

        @pl.when(s == 15)
        def _():
            for j in range(3):
                _dfire(15 * ZCH + j * WIN, WIN)
            _dfire(15 * ZCH + 3 * WIN, 16)
            for j in range(3):
                _ddrain(15 * ZCH + j * WIN, WIN)
            _ddrain(15 * ZCH + 3 * WIN, 16)

    return segsum


def _row_spec(d):
    return pl.BlockSpec((ROWB, d), lambda i: (i, 0))


def _seg_spec(d):
    return pl.BlockSpec((2, ROWB, d), lambda i: (0, i, 0))


_CNT_SPEC = pl.BlockSpec((2, 1, 1, ROWB), lambda i: (0, i, 0, 0))


def _full_spec(shape):
    rank = len(shape)
    return pl.BlockSpec(shape, lambda i: (0,) * rank)


def _tc_call(body, in_arrays, in_specs, out_d):
    if isinstance(out_d, (tuple, list)):
        out_shape = tuple(jax.ShapeDtypeStruct((N, dd), jnp.float32)
                          for dd in out_d)
        out_specs = tuple(_row_spec(dd) for dd in out_d)
    else:
        out_shape = jax.ShapeDtypeStruct((N, out_d), jnp.float32)
        out_specs = _row_spec(out_d)
    return pl.pallas_call(
        body,
        grid=(N // ROWB,),
        in_specs=in_specs,
        out_specs=out_specs,
        out_shape=out_shape,
    )(*in_arrays)


def _invc(cnt_ref):
    cnt = cnt_ref[0, 0, 0, :] + cnt_ref[1, 0, 0, :]
    return 1.0 / jnp.maximum(cnt, 1.0)


def _mean(seg_ref, cnt_ref):
    ssum = seg_ref[0] + seg_ref[1]
    return ssum * _invc(cnt_ref)[:, None]


def _stage_a(x_r, w_r, b_r, o_r):
    x0 = jnp.tanh(_dg(x_r[...], w_r[...]) + b_r[...])
    o_r[...] = jnp.concatenate(
        [x0, jnp.ones((ROWB, 16), jnp.float32)], axis=1)


def _stage_b(s_r, c_r, x_r, wl_r, bl_r, wr_r, o_r):
    ssum = s_r[0, :, :32] + s_r[1, :, :32]
    mean = ssum * _invc(c_r)[:, None]
    o_r[...] = jnp.tanh(_dg(mean, wl_r[...]) + bl_r[...] +
                        _dg(x_r[:, :32], wr_r[...]))


def _stage_sage(s_r, c_r, x_r, wl_r, bl_r, wr_r, o_r):
    mean = _mean(s_r, c_r)
    o_r[...] = jnp.tanh(_dg(mean, wl_r[...]) + bl_r[...] +
                        _dg(x_r[...], wr_r[...]))


def _stage_d(s_r, c_r, x2_r, bnwl_r, bnbl_r, bnwr_r, d1wl_r, x1_r,
             s1w_r, s1b_r, x3_r, g3_r, sk1_r):
    invc = _invc(c_r)[:, None]
    mean_a = s_r[0] * invc
    mean_b = s_r[1] * invc
    x3 = jnp.tanh(_dg(mean_a, bnwl_r[:, :64]) + _dg(mean_b, bnwl_r[:, 64:]) +
                  bnbl_r[...] + _dg(x2_r[...], bnwr_r[...]))
    x3_r[...] = x3
    g3_r[...] = _dg(x3, d1wl_r[:, :128]) + _dg(x2_r[...], d1wl_r[:, 128:])
    sk1_r[...] = _dg(x1_r[...], s1w_r[...]) + s1b_r[...]


def _stage_e(s_r, c_r, x3_r, x2_r, d1bl_r, d1wr_r, sk1_r, d2wl_r,
             h4_r, g4_r):
    mean = _mean(s_r, c_r)
    h4 = jnp.tanh(mean + d1bl_r[...] + _dg(x3_r[...], d1wr_r[:, :128]) +
                  _dg(x2_r[...], d1wr_r[:, 128:]))
    h4_r[...] = h4
    g4_r[...] = _dg(h4, d2wl_r[:, :64]) + _dg(sk1_r[...], d2wl_r[:, 64:])


def _stage_f(s_r, c_r, h4_r, sk1_r, d2bl_r, d2wr_r, x0_r, s2w_r, s2b_r,
             outw_r, outb_r, o_r):
    mean = _mean(s_r, c_r)
    h5 = jnp.tanh(mean + d2bl_r[...] + _dg(h4_r[...], d2wr_r[:, :64]) +
                  _dg(sk1_r[...], d2wr_r[:, 64:]))
    skip2 = _dg(x0_r[:, :32], s2w_r[...]) + s2b_r[...]
    o_r[...] = (_dg(h5, outw_r[:, :32]) + _dg(skip2, outw_r[:, 32:]) +
                outb_r[...])


def kernel(x, edge_index, fc_W, fc_b, c1_Wl, c1_bl, c1_Wr, c2_Wl, c2_bl,
           c2_Wr, bn_Wl, bn_bl, bn_Wr, d1_Wl, d1_bl, d1_Wr, d2_Wl, d2_bl,
           d2_Wr, out_W, out_b, s1_W, s1_b, s2_W, s2_b):
    pad = EPAD - E
    pada = jnp.arange(pad, dtype=jnp.int32)
    srcw = jnp.concatenate([edge_index[0], pada % 256]).reshape(-1, WIN)
    dstw = jnp.concatenate([edge_index[1], N + (pada % NJUNK)]).reshape(-1, WIN)

    r = lambda b: b.reshape(1, -1)

    x0a = _tc_call(
        _stage_a, (x, fc_W, r(fc_b)),
        [_row_spec(128), _full_spec((32, 128)), _full_spec((1, 32))], 48)

    s0 = _segsum(x0a, srcw, dstw)
    cnt = s0[:, :, 32].reshape(2, N // ROWB, 1, ROWB)
    x1 = _tc_call(
        _stage_b, (s0, cnt, x0a, c1_Wl, r(c1_bl), c1_Wr),
        [_seg_spec(48), _CNT_SPEC, _row_spec(48), _full_spec((64, 32)),
         _full_spec((1, 64)), _full_spec((64, 32))], 64)

    s1, = (_segsum(x1, srcw, dstw),)
    x2 = _tc_call(
        _stage_sage, (s1, cnt, x1, c2_Wl, r(c2_bl), c2_Wr),
        [_seg_spec(64), _CNT_SPEC, _row_spec(64), _full_spec((128, 64)),
         _full_spec((1, 128)), _full_spec((128, 64))], 128)

    s2m = _make_segsum_colsplit()(
        x2[:, :64], x2[:, 64:], srcw, dstw,
        jnp.zeros((WIN, 64), jnp.float32))
    x3, g3, skip1 = _tc_call(
        _stage_d, (s2m, cnt, x2, bn_Wl, r(bn_bl), bn_Wr, d1_Wl, x1,
                   s1_W, r(s1_b)),
        [_seg_spec(64), _CNT_SPEC, _row_spec(128),
         _full_spec((128, 128)), _full_spec((1, 128)),
         _full_spec((128, 128)), _full_spec((64, 256)),
         _row_spec(64), _full_spec((64, 64)), _full_spec((1, 64))],
        (128, 64, 64))

    s3 = _segsum(g3, srcw, dstw)
    h4, g4 = _tc_call(
        _stage_e, (s3, cnt, x3, x2, r(d1_bl), d1_Wr, skip1, d2_Wl),
        [_seg_spec(64), _CNT_SPEC, _row_spec(128), _row_spec(128),
         _full_spec((1, 64)), _full_spec((64, 256)), _row_spec(64),
         _full_spec((32, 128))], (64, 32))

    s4 = _segsum(g4, srcw, dstw)
    o = _tc_call(
        _stage_f, (s4, cnt, h4, skip1, r(d2_bl), d2_Wr, x0a, s2_W, r(s2_b),
                   out_W, r(out_b)),
        [_seg_spec(32), _CNT_SPEC, _row_spec(64), _row_spec(64),
         _full_spec((1, 32)), _full_spec((32, 128)), _row_spec(48),
         _full_spec((32, 32)), _full_spec((1, 32)), _full_spec((3, 64)),
         _full_spec((1, 3))], 3)

    return o
